# baseline (speedup 1.0000x reference)
import functools

import jax
import jax.numpy as jnp
import numpy as np
from jax.experimental import pallas as pl
from jax.experimental.pallas import tpu as pltpu

N_PHASES = 10
SPARSITY_RATIO = 0.23
VOCAB_SIZE = 100000
TOPK = 16
V_BLK = 2000
NEG = -1e30
INV_MAX_ENT = float(np.log(VOCAB_SIZE) + 1e-09)


def _entropy_mask_kernel(phase_ref, idx_ref, x_ref, o_ref, m_scr, se_scr,
                         sxe_scr, *, num_v, t):
    v = pl.program_id(0)

    @pl.when(v == 0)
    def _init():
        m_scr[...] = jnp.full((1, t), NEG, jnp.float32)
        se_scr[...] = jnp.zeros((1, t), jnp.float32)
        sxe_scr[...] = jnp.zeros((1, t), jnp.float32)

    x = x_ref[0]
    m_old = m_scr[...]
    m_new = jnp.maximum(m_old, jnp.max(x, axis=0, keepdims=True))
    alpha = jnp.exp(m_old - m_new)
    e = jnp.exp(x - m_new)
    ones = jnp.ones((1, x.shape[0]), jnp.float32)
    se_blk = jax.lax.dot_general(
        ones, e, (((1,), (0,)), ((), ())),
        precision=jax.lax.Precision.HIGHEST,
        preferred_element_type=jnp.float32)
    sxe_blk = jax.lax.dot_general(
        ones, x * e, (((1,), (0,)), ((), ())),
        precision=jax.lax.Precision.HIGHEST,
        preferred_element_type=jnp.float32)
    se_new = se_scr[...] * alpha + se_blk
    sxe_new = sxe_scr[...] * alpha + sxe_blk
    m_scr[...] = m_new
    se_scr[...] = se_new
    sxe_scr[...] = sxe_new

    @pl.when(v == num_v - 1)
    def _finalize():
        ent = (m_new + jnp.log(se_new)) - sxe_new / se_new
        ent_mean = jnp.sum(ent) / np.float32(t) / np.float32(INV_MAX_ENT)
        ent_factor = jnp.clip(ent_mean, 0.0, 1.0) * 0.5
        base = np.float32(max(1e-06, 1.0 - SPARSITY_RATIO))
        ratio = jnp.clip(base + ent_factor, 0.05, 1.0)
        active = jnp.clip(jnp.round(np.float32(t) * ratio), 1, t).astype(
            jnp.int32)
        max_start = jnp.maximum(0, t - active)
        step = jnp.maximum(1, active // 2)
        phase = phase_ref[0]
        start = (phase * step) % (max_start + 1)

        pos = jax.lax.broadcasted_iota(jnp.int32, (1, t), 1)
        window = (pos >= start) & (pos < start + active)
        cond = window | (active >= t) | (phase >= N_PHASES - 1)
        for k in range(TOPK):
            idx_k = idx_ref[k]
            cond = cond | ((pos == idx_k) & (idx_k < t))
        o_ref[...] = jnp.where(cond, 1.0, 0.0).astype(jnp.float32)


def kernel(input_ids, logits, phase, last_phase_top_indices):
    del input_ids
    b, t, vocab = logits.shape
    xt = jnp.swapaxes(logits, 1, 2)
    num_v = vocab // V_BLK
    phase_arr = jnp.asarray(phase, jnp.int32).reshape(1)
    idx_arr = last_phase_top_indices.astype(jnp.int32).reshape(TOPK)

    grid_spec = pltpu.PrefetchScalarGridSpec(
        num_scalar_prefetch=2,
        grid=(num_v,),
        in_specs=[
            pl.BlockSpec((1, V_BLK, t), lambda v, *_: (0, v, 0)),
        ],
        out_specs=pl.BlockSpec((1, t), lambda v, *_: (0, 0)),
        scratch_shapes=[
            pltpu.VMEM((1, t), jnp.float32),
            pltpu.VMEM((1, t), jnp.float32),
            pltpu.VMEM((1, t), jnp.float32),
        ],
    )
    out = pl.pallas_call(
        functools.partial(_entropy_mask_kernel, num_v=num_v, t=t),
        grid_spec=grid_spec,
        out_shape=jax.ShapeDtypeStruct((1, t), jnp.float32),
        compiler_params=pltpu.CompilerParams(
            dimension_semantics=("arbitrary",)),
    )(phase_arr, idx_arr, xt)
    return out


# speedup vs baseline: 2.4753x; 2.4753x over previous
import functools

import jax
import jax.numpy as jnp
import numpy as np
from jax.experimental import pallas as pl
from jax.experimental.pallas import tpu as pltpu

N_PHASES = 10
SPARSITY_RATIO = 0.23
VOCAB_SIZE = 100000
TOPK = 16
V_BLK = 2000
NEG = -1e30
INV_MAX_ENT = float(np.log(VOCAB_SIZE) + 1e-09)


def _entropy_mask_kernel(phase_ref, idx_ref, x_ref, o_ref, m_scr, se_scr,
                         sxe_scr, *, num_v, t):
    v = pl.program_id(0)

    @pl.when(v == 0)
    def _init():
        m_scr[...] = jnp.full((1, t), NEG, jnp.float32)
        se_scr[...] = jnp.zeros((1, t), jnp.float32)
        sxe_scr[...] = jnp.zeros((1, t), jnp.float32)

    x = x_ref[0]
    m_old = m_scr[...]
    m_new = jnp.maximum(m_old, jnp.max(x, axis=0, keepdims=True))
    alpha = jnp.exp(m_old - m_new)
    e = jnp.exp(x - m_new)
    se_new = se_scr[...] * alpha + jnp.sum(e, axis=0, keepdims=True)
    sxe_new = sxe_scr[...] * alpha + jnp.sum(x * e, axis=0, keepdims=True)
    m_scr[...] = m_new
    se_scr[...] = se_new
    sxe_scr[...] = sxe_new

    @pl.when(v == num_v - 1)
    def _finalize():
        ent = (m_new + jnp.log(se_new)) - sxe_new / se_new
        ent_mean = jnp.sum(ent) / np.float32(t) / np.float32(INV_MAX_ENT)
        ent_factor = jnp.clip(ent_mean, 0.0, 1.0) * 0.5
        base = np.float32(max(1e-06, 1.0 - SPARSITY_RATIO))
        ratio = jnp.clip(base + ent_factor, 0.05, 1.0)
        active = jnp.clip(jnp.round(np.float32(t) * ratio), 1, t).astype(
            jnp.int32)
        max_start = jnp.maximum(0, t - active)
        step = jnp.maximum(1, active // 2)
        phase = phase_ref[0]
        start = (phase * step) % (max_start + 1)

        pos = jax.lax.broadcasted_iota(jnp.int32, (1, t), 1)
        window = (pos >= start) & (pos < start + active)
        cond = window | (active >= t) | (phase >= N_PHASES - 1)
        for k in range(TOPK):
            idx_k = idx_ref[k]
            cond = cond | ((pos == idx_k) & (idx_k < t))
        o_ref[...] = jnp.where(cond, 1.0, 0.0).astype(jnp.float32)


def kernel(input_ids, logits, phase, last_phase_top_indices):
    del input_ids
    b, t, vocab = logits.shape
    xt = jnp.swapaxes(logits, 1, 2)
    num_v = vocab // V_BLK
    phase_arr = jnp.asarray(phase, jnp.int32).reshape(1)
    idx_arr = last_phase_top_indices.astype(jnp.int32).reshape(TOPK)

    grid_spec = pltpu.PrefetchScalarGridSpec(
        num_scalar_prefetch=2,
        grid=(num_v,),
        in_specs=[
            pl.BlockSpec((1, V_BLK, t), lambda v, *_: (0, v, 0)),
        ],
        out_specs=pl.BlockSpec((1, t), lambda v, *_: (0, 0)),
        scratch_shapes=[
            pltpu.VMEM((1, t), jnp.float32),
            pltpu.VMEM((1, t), jnp.float32),
            pltpu.VMEM((1, t), jnp.float32),
        ],
    )
    out = pl.pallas_call(
        functools.partial(_entropy_mask_kernel, num_v=num_v, t=t),
        grid_spec=grid_spec,
        out_shape=jax.ShapeDtypeStruct((1, t), jnp.float32),
        compiler_params=pltpu.CompilerParams(
            dimension_semantics=("arbitrary",)),
    )(phase_arr, idx_arr, xt)
    return out
